# baseline (speedup 1.0000x reference)
import functools

import jax
import jax.numpy as jnp
from jax import lax
from jax.experimental import pallas as pl
from jax.experimental.pallas import tpu as pltpu
from jax.experimental.pallas import tpu_sc as plsc

B, L, D = 4096, 200, 32
V = 1000000
NC, NS = 2, 16
NW = NC * NS
EPW = B // NW
IW = 40
NCHUNK = EPW * L // IW
RING = 5
CPE = L // IW
RPW = EPW * L // IW
RPC = IW
TW = 128
INV_L = 1.0 / L

BLK = 32768
NBLK = -(-V // BLK)

_mesh = plsc.VectorSubcoreMesh(core_axis_name="c", subcore_axis_name="s")


def _widen_body(x_ref, o_ref):
    x = x_ref[...]
    qf = lax.broadcasted_iota(jnp.int32, (32, TW), 1) % 32
    ff = lax.broadcasted_iota(jnp.int32, (32, TW), 0)
    rep = (qf == ff).astype(jnp.float32)
    o_ref[...] = lax.dot_general(x, rep, (((0,), (0,)), ((), ())),
                                 preferred_element_type=jnp.float32)


_widen = pl.pallas_call(
    _widen_body,
    grid=(NBLK,),
    in_specs=[pl.BlockSpec((32, BLK), lambda i: (0, i))],
    out_specs=pl.BlockSpec((BLK, TW), lambda i: (i, 0)),
    out_shape=jax.ShapeDtypeStruct((V, TW), jnp.float32),
)


@functools.partial(
    pl.kernel,
    out_type=jax.ShapeDtypeStruct((B, D), jnp.float32),
    mesh=_mesh,
    compiler_params=pltpu.CompilerParams(use_tc_tiling_on_sc=True),
    scratch_types=[
        pltpu.VMEM((RPW, IW), jnp.int32),
        pltpu.VMEM((RING, RPC, TW), jnp.float32),
        pltpu.VMEM((EPW, D), jnp.float32),
    ] + [pltpu.SemaphoreType.DMA] * RING,
)
def _embed_mean(idx_hbm, table_hbm, out_hbm, idx_v, rows_v, out_v, *sems):
    wid = lax.axis_index("s") * NC + lax.axis_index("c")
    elem0 = wid * EPW

    pltpu.sync_copy(idx_hbm.at[pl.ds(wid * RPW, RPW)], idx_v)

    def issue(c, rows, sem):
        pltpu.async_copy(table_hbm.at[idx_v.at[c]], rows, sem)

    def drain(rows, sem):
        pltpu.make_async_copy(table_hbm.at[pl.ds(0, RPC)], rows, sem).wait()

    def reduce_store(c, rows):
        def red(r, acc):
            a0, a1, b0, b1 = acc
            row = 2 * r
            a0 = a0 + rows[row, pl.ds(0, 16)]
            a1 = a1 + rows[row, pl.ds(16, 16)]
            b0 = b0 + rows[row + 1, pl.ds(0, 16)]
            b1 = b1 + rows[row + 1, pl.ds(16, 16)]
            return (a0, a1, b0, b1)

        z = jnp.zeros((16,), jnp.float32)
        a0, a1, b0, b1 = lax.fori_loop(0, RPC // 2, red, (z, z, z, z),
                                       unroll=10)
        el = lax.div(c, CPE)
        out_v[el, pl.ds(0, 16)] = out_v[el, pl.ds(0, 16)] + (a0 + b0)
        out_v[el, pl.ds(16, 16)] = out_v[el, pl.ds(16, 16)] + (a1 + b1)

    def zero_body(e, carry):
        z = jnp.zeros((16,), jnp.float32)
        out_v[e, pl.ds(0, 16)] = z
        out_v[e, pl.ds(16, 16)] = z
        return carry

    lax.fori_loop(0, EPW, zero_body, 0)

    for k in range(RING):
        issue(k, rows_v.at[k], sems[k])

    def ring_body(i, carry):
        c = RING * i
        for k in range(RING):
            drain(rows_v.at[k], sems[k])
            reduce_store(c + k, rows_v.at[k])
            issue(c + RING + k, rows_v.at[k], sems[k])
        return carry

    lax.fori_loop(0, NCHUNK // RING - 1, ring_body, 0)

    for k in range(RING):
        drain(rows_v.at[k], sems[k])
        reduce_store(NCHUNK - RING + k, rows_v.at[k])

    def scale_body(e, carry):
        out_v[e, pl.ds(0, 16)] = out_v[e, pl.ds(0, 16)] * INV_L
        out_v[e, pl.ds(16, 16)] = out_v[e, pl.ds(16, 16)] * INV_L
        return carry

    lax.fori_loop(0, EPW, scale_body, 0)

    pltpu.sync_copy(out_v, out_hbm.at[pl.ds(elem0, EPW)])


def kernel(input, table):
    idx2 = input.astype(jnp.int32).reshape(B * L // IW, IW)
    wide = _widen(table.T)
    return _embed_mean(idx2, wide)
